# baseline (speedup 1.0000x reference)
import functools

import jax
import jax.numpy as jnp
from jax import lax
from jax.experimental import pallas as pl
from jax.experimental.pallas import tpu as pltpu
from jax.experimental.pallas import tpu_sc as plsc

_N = 10000
_D = 128
_E = 320000
_EPS = 1e-5

_NC = 2
_NS = 16
_NW = _NC * _NS
_CHUNK = 128
_KCH = 80
_PH = 40
_EPAD = _NW * _KCH * _CHUNK
_NACC = 10240
_RPT = _NACC // _NS
_NB = 10
_BR = _N // _NB

@functools.cache
def _sc_kernels():
    mesh = plsc.VectorSubcoreMesh(core_axis_name="c", subcore_axis_name="s")

    @functools.partial(
        pl.kernel,
        out_type=jax.ShapeDtypeStruct((_NC, _NACC, _D), jnp.float32),
        mesh=mesh,
        scratch_types=[
            pltpu.VMEM((_PH, _CHUNK), jnp.int32),
            pltpu.VMEM((_PH, _CHUNK), jnp.int32),
            pltpu.VMEM((2, _CHUNK, _D), jnp.float32),
            pltpu.VMEM_SHARED((_NACC, _D), jnp.float32),
            pltpu.SemaphoreType.DMA,
            pltpu.SemaphoreType.DMA,
            pltpu.SemaphoreType.DMA,
            pltpu.SemaphoreType.DMA,
        ],
    )
    def sc_scatter(hs, src2, dst2, zpad, t, isa, ida, rows2, acc, g0, g1,
                   s0, s1):
        c = lax.axis_index("c")
        s = lax.axis_index("s")
        wid = c * _NS + s
        row0 = s * _RPT

        def gat(j, sl, sem):
            pltpu.async_copy(hs.at[isa.at[j]], rows2.at[sl], sem)

        def gwait(sl, sem):
            pltpu.make_async_copy(hs.at[isa.at[0]], rows2.at[sl], sem).wait()

        def sca(j, sl, sem):
            pltpu.async_copy(rows2.at[sl], acc.at[ida.at[j]], sem, add=True)

        def swait(sl, sem):
            pltpu.make_async_copy(rows2.at[sl], acc.at[ida.at[0]],
                                  sem).wait()

        pltpu.sync_copy(zpad, acc.at[pl.ds(row0, _RPT)])
        plsc.subcore_barrier()

        for p in range(_KCH // _PH):
            base = wid * _KCH + p * _PH
            pltpu.sync_copy(src2.at[pl.ds(base, _PH)], isa)
            pltpu.sync_copy(dst2.at[pl.ds(base, _PH)], ida)
            gat(0, 0, g0)
            gat(1, 1, g1)
            gwait(0, g0)
            sca(0, 0, s0)

            def body2(i, carry):
                j = 2 * i + 1
                gwait(1, g1)
                sca(j, 1, s1)
                swait(0, s0)
                gat(j + 1, 0, g0)
                gwait(0, g0)
                sca(j + 1, 0, s0)
                swait(1, s1)
                gat(j + 2, 1, g1)
                return carry

            lax.fori_loop(0, _PH // 2 - 1, body2, 0)
            gwait(1, g1)
            sca(_PH - 1, 1, s1)
            swait(0, s0)
            swait(1, s1)

        plsc.subcore_barrier()
        pltpu.sync_copy(acc.at[pl.ds(row0, _RPT)], t.at[c, pl.ds(row0, _RPT)])

    @functools.partial(
        pl.kernel,
        out_type=jax.ShapeDtypeStruct((_NC, _NACC, _D), jnp.float32),
        mesh=mesh,
        scratch_types=[
            pltpu.VMEM((_PH, _CHUNK), jnp.int32),
            pltpu.VMEM((_CHUNK, _D), jnp.float32),
            pltpu.VMEM_SHARED((_NACC, _D), jnp.float32),
            pltpu.SemaphoreType.DMA,
        ],
    )
    def sc_count(ones_nd, dst2, zpad, tdeg, ida, rows, acc, sem):
        c = lax.axis_index("c")
        s = lax.axis_index("s")
        wid = c * _NS + s
        row0 = s * _RPT
        pltpu.sync_copy(ones_nd.at[pl.ds(0, _CHUNK)], rows)
        pltpu.sync_copy(zpad, acc.at[pl.ds(row0, _RPT)])
        plsc.subcore_barrier()

        for p in range(_KCH // _PH):
            pltpu.sync_copy(dst2.at[pl.ds(wid * _KCH + p * _PH, _PH)], ida)

            def fire(j, carry):
                pltpu.async_copy(rows, acc.at[ida.at[j]], sem, add=True)
                return carry

            lax.fori_loop(0, _PH, fire, 0)

            def drain(j, carry):
                pltpu.make_async_copy(rows, acc.at[ida.at[0]], sem).wait()
                return carry

            lax.fori_loop(0, _PH, drain, 0)

        plsc.subcore_barrier()
        pltpu.sync_copy(acc.at[pl.ds(row0, _RPT)], tdeg.at[c, pl.ds(row0, _RPT)])

    return sc_scatter, sc_count


_MM = dict(preferred_element_type=jnp.float32, precision=lax.Precision.HIGHEST)


def _tc_prep_body(d0, d1, x, w, hs, dinv):
    deg = d0[...] + d1[...] + 1.0
    di = lax.rsqrt(deg)
    dinv[...] = di
    hs[...] = jnp.dot(x[...], w[...], **_MM) * di


def _tc_stats_body(t0, t1, hs, dinv, b, agg, sums):
    i = pl.program_id(0)
    a = (t0[...] + t1[...] + hs[...]) * dinv[...] + b[...]
    agg[...] = a

    @pl.when(i == 0)
    def _():
        sums[...] = jnp.zeros_like(sums)

    sums[0:1, :] += jnp.sum(a, axis=0, keepdims=True)
    sums[1:2, :] += jnp.sum(a * a, axis=0, keepdims=True)


def _bn_relu(agg_ref, sums_ref, g_ref, be_ref):
    mean = sums_ref[0:1, :] * (1.0 / _N)
    var = sums_ref[1:2, :] * (1.0 / _N) - mean * mean
    inv = lax.rsqrt(var + _EPS)
    return jnp.maximum((agg_ref[...] - mean) * inv * g_ref[...] + be_ref[...], 0.0)


def _tc_norm_mm_body(agg, sums, g, be, w, dinv, out):
    y = _bn_relu(agg, sums, g, be)
    out[...] = jnp.dot(y, w[...], **_MM) * dinv[...]


def _tc_norm_final_body(agg, sums, g, be, out):
    out[...] = _bn_relu(agg, sums, g, be)


_blk = lambda r, c: pl.BlockSpec((r, c), lambda i: (i, 0))
_bcast = lambda r, c: pl.BlockSpec((r, c), lambda i: (0, 0))

_tc_prep = pl.pallas_call(
    _tc_prep_body,
    grid=(_NB,),
    in_specs=[_blk(_BR, _D), _blk(_BR, _D), _blk(_BR, _D), _bcast(_D, _D)],
    out_specs=[_blk(_BR, _D), _blk(_BR, _D)],
    out_shape=[
        jax.ShapeDtypeStruct((_N, _D), jnp.float32),
        jax.ShapeDtypeStruct((_N, _D), jnp.float32),
    ],
)

_tc_stats = pl.pallas_call(
    _tc_stats_body,
    grid=(_NB,),
    in_specs=[_blk(_BR, _D), _blk(_BR, _D), _blk(_BR, _D), _blk(_BR, _D),
              _bcast(1, _D)],
    out_specs=[_blk(_BR, _D), _bcast(8, _D)],
    out_shape=[
        jax.ShapeDtypeStruct((_N, _D), jnp.float32),
        jax.ShapeDtypeStruct((8, _D), jnp.float32),
    ],
)

_tc_norm_mm = pl.pallas_call(
    _tc_norm_mm_body,
    grid=(_NB,),
    in_specs=[_blk(_BR, _D), _bcast(8, _D), _bcast(1, _D), _bcast(1, _D),
              _bcast(_D, _D), _blk(_BR, _D)],
    out_specs=_blk(_BR, _D),
    out_shape=jax.ShapeDtypeStruct((_N, _D), jnp.float32),
)

_tc_norm_final = pl.pallas_call(
    _tc_norm_final_body,
    grid=(_NB,),
    in_specs=[_blk(_BR, _D), _bcast(8, _D), _bcast(1, _D), _bcast(1, _D)],
    out_specs=_blk(_BR, _D),
    out_shape=jax.ShapeDtypeStruct((_N, _D), jnp.float32),
)


def kernel(x, edge_index, W0, b0, g0, be0, W1, b1, g1, be1, W2, b2, g2, be2):
    src = edge_index[0]
    dst = edge_index[1]
    pad = _EPAD - _E
    spread = (jnp.arange(pad, dtype=jnp.int32) % 128)
    src2 = jnp.concatenate([src, spread]).reshape(_NW * _KCH, _CHUNK)
    dst2 = jnp.concatenate([dst, _N + spread]).reshape(_NW * _KCH, _CHUNK)
    zpad = jnp.zeros((_RPT, _D), jnp.float32)

    sc_scatter, sc_count = _sc_kernels()
    ones_nd = jnp.ones((_N, _D), jnp.float32)
    tdeg = sc_count(ones_nd, dst2, zpad)
    hs, dinv = _tc_prep(tdeg[0], tdeg[1], x, W0)

    params = ((b0, g0, be0, W1), (b1, g1, be1, W2), (b2, g2, be2, None))
    for b, g, be, Wn in params:
        t = sc_scatter(hs, src2, dst2, zpad)
        agg, sums = _tc_stats(t[0], t[1], hs, dinv, b.reshape(1, _D))
        if Wn is not None:
            hs = _tc_norm_mm(agg, sums, g.reshape(1, _D), be.reshape(1, _D),
                             Wn, dinv)
        else:
            out = _tc_norm_final(agg, sums, g.reshape(1, _D), be.reshape(1, _D))
    return out


# speedup vs baseline: 1.1479x; 1.1479x over previous
import functools

import jax
import jax.numpy as jnp
from jax import lax
from jax.experimental import pallas as pl
from jax.experimental.pallas import tpu as pltpu
from jax.experimental.pallas import tpu_sc as plsc

_N = 10000
_D = 128
_E = 320000
_EPS = 1e-5

_NC = 2
_NS = 16
_NW = _NC * _NS
_CHUNK = 128
_KCH = 80
_PH = 40
_EPAD = _NW * _KCH * _CHUNK
_NACC = 10240
_RPT = _NACC // _NS
_NB = 10
_BR = _N // _NB

@functools.cache
def _sc_kernels():
    mesh = plsc.VectorSubcoreMesh(core_axis_name="c", subcore_axis_name="s")

    @functools.partial(
        pl.kernel,
        out_type=jax.ShapeDtypeStruct((_NC, _NACC, _D), jnp.float32),
        mesh=mesh,
        scratch_types=[
            pltpu.VMEM((_PH, _CHUNK), jnp.int32),
            pltpu.VMEM((_PH, _CHUNK), jnp.int32),
            pltpu.VMEM((2, _CHUNK, _D), jnp.float32),
            pltpu.VMEM_SHARED((_NACC, _D), jnp.float32),
            pltpu.SemaphoreType.DMA,
            pltpu.SemaphoreType.DMA,
        ],
    )
    def sc_scatter(hs, src2, dst2, zpad, t, isa, ida, rows2, acc, g0, g1):
        c = lax.axis_index("c")
        s = lax.axis_index("s")
        wid = c * _NS + s
        row0 = s * _RPT
        pltpu.sync_copy(zpad, acc.at[pl.ds(row0, _RPT)])
        plsc.subcore_barrier()

        for p in range(_KCH // _PH):
            base = wid * _KCH + p * _PH
            pltpu.sync_copy(src2.at[pl.ds(base, _PH)], isa)
            pltpu.sync_copy(dst2.at[pl.ds(base, _PH)], ida)
            pltpu.async_copy(hs.at[isa.at[0]], rows2.at[0], g0)

            def body2(i, carry):
                j = 2 * i
                pltpu.async_copy(hs.at[isa.at[j + 1]], rows2.at[1], g1)
                pltpu.make_async_copy(hs.at[isa.at[0]], rows2.at[0], g0).wait()
                pltpu.sync_copy(rows2.at[0], acc.at[ida.at[j]], add=True)
                pltpu.async_copy(hs.at[isa.at[j + 2]], rows2.at[0], g0)
                pltpu.make_async_copy(hs.at[isa.at[0]], rows2.at[1], g1).wait()
                pltpu.sync_copy(rows2.at[1], acc.at[ida.at[j + 1]], add=True)
                return carry

            lax.fori_loop(0, _PH // 2 - 1, body2, 0)
            pltpu.async_copy(hs.at[isa.at[_PH - 1]], rows2.at[1], g1)
            pltpu.make_async_copy(hs.at[isa.at[0]], rows2.at[0], g0).wait()
            pltpu.sync_copy(rows2.at[0], acc.at[ida.at[_PH - 2]], add=True)
            pltpu.make_async_copy(hs.at[isa.at[0]], rows2.at[1], g1).wait()
            pltpu.sync_copy(rows2.at[1], acc.at[ida.at[_PH - 1]], add=True)

        plsc.subcore_barrier()
        pltpu.sync_copy(acc.at[pl.ds(row0, _RPT)], t.at[c, pl.ds(row0, _RPT)])

    @functools.partial(
        pl.kernel,
        out_type=jax.ShapeDtypeStruct((_NC, _NACC, _D), jnp.float32),
        mesh=mesh,
        scratch_types=[
            pltpu.VMEM((_PH, _CHUNK), jnp.int32),
            pltpu.VMEM((_CHUNK, _D), jnp.float32),
            pltpu.VMEM_SHARED((_NACC, _D), jnp.float32),
            pltpu.SemaphoreType.DMA,
        ],
    )
    def sc_count(ones_nd, dst2, zpad, tdeg, ida, rows, acc, sem):
        c = lax.axis_index("c")
        s = lax.axis_index("s")
        wid = c * _NS + s
        row0 = s * _RPT
        pltpu.sync_copy(ones_nd.at[pl.ds(0, _CHUNK)], rows)
        pltpu.sync_copy(zpad, acc.at[pl.ds(row0, _RPT)])
        plsc.subcore_barrier()

        for p in range(_KCH // _PH):
            pltpu.sync_copy(dst2.at[pl.ds(wid * _KCH + p * _PH, _PH)], ida)

            def fire(j, carry):
                pltpu.async_copy(rows, acc.at[ida.at[j]], sem, add=True)
                return carry

            lax.fori_loop(0, _PH, fire, 0)

            def drain(j, carry):
                pltpu.make_async_copy(rows, acc.at[ida.at[0]], sem).wait()
                return carry

            lax.fori_loop(0, _PH, drain, 0)

        plsc.subcore_barrier()
        pltpu.sync_copy(acc.at[pl.ds(row0, _RPT)], tdeg.at[c, pl.ds(row0, _RPT)])

    return sc_scatter, sc_count


_MM = dict(preferred_element_type=jnp.float32, precision=lax.Precision.HIGHEST)


def _tc_prep_body(d0, d1, x, w, hs, dinv):
    deg = d0[...] + d1[...] + 1.0
    di = lax.rsqrt(deg)
    dinv[...] = di
    hs[...] = jnp.dot(x[...], w[...], **_MM) * di


def _tc_layer_body(with_mm, t0, t1, hs, dinv, b, g, be, w, out, agg_s,
                   dinv_s, sums_s):
    p = pl.program_id(0)
    i = pl.program_id(1)

    @pl.when(p == 0)
    def _():
        a = (t0[...] + t1[...] + hs[...]) * dinv[...] + b[...]
        agg_s[pl.ds(i * _BR, _BR)] = a
        dinv_s[pl.ds(i * _BR, _BR)] = dinv[...]

        @pl.when(i == 0)
        def _():
            sums_s[...] = jnp.zeros_like(sums_s)

        sums_s[0:1, :] += jnp.sum(a, axis=0, keepdims=True)
        sums_s[1:2, :] += jnp.sum(a * a, axis=0, keepdims=True)

    @pl.when(p == 1)
    def _():
        mean = sums_s[0:1, :] * (1.0 / _N)
        var = sums_s[1:2, :] * (1.0 / _N) - mean * mean
        inv = lax.rsqrt(var + _EPS)
        a = agg_s[pl.ds(i * _BR, _BR)]
        y = jnp.maximum((a - mean) * inv * g[...] + be[...], 0.0)
        if with_mm:
            out[...] = jnp.dot(y, w[...], **_MM) * dinv_s[pl.ds(i * _BR, _BR)]
        else:
            out[...] = y


_blk = lambda r, c: pl.BlockSpec((r, c), lambda i: (i, 0))
_bcast = lambda r, c: pl.BlockSpec((r, c), lambda i: (0, 0))

_tc_prep = pl.pallas_call(
    _tc_prep_body,
    grid=(_NB,),
    in_specs=[_blk(_BR, _D), _blk(_BR, _D), _blk(_BR, _D), _bcast(_D, _D)],
    out_specs=[_blk(_BR, _D), _blk(_BR, _D)],
    out_shape=[
        jax.ShapeDtypeStruct((_N, _D), jnp.float32),
        jax.ShapeDtypeStruct((_N, _D), jnp.float32),
    ],
)

_pblk = pl.BlockSpec((_BR, _D), lambda p, i: (i * (1 - p), 0))
_pbc1 = pl.BlockSpec((1, _D), lambda p, i: (0, 0))
_pbcw = pl.BlockSpec((_D, _D), lambda p, i: (0, 0))
_pout = pl.BlockSpec((_BR, _D), lambda p, i: (i * p, 0))
_layer_scratch = [
    pltpu.VMEM((_N, _D), jnp.float32),
    pltpu.VMEM((_N, _D), jnp.float32),
    pltpu.VMEM((8, _D), jnp.float32),
]

_tc_layer_mm = pl.pallas_call(
    functools.partial(_tc_layer_body, True),
    grid=(2, _NB),
    in_specs=[_pblk, _pblk, _pblk, _pblk, _pbc1, _pbc1, _pbc1, _pbcw],
    out_specs=_pout,
    out_shape=jax.ShapeDtypeStruct((_N, _D), jnp.float32),
    scratch_shapes=_layer_scratch,
)

_tc_layer_final = pl.pallas_call(
    functools.partial(_tc_layer_body, False),
    grid=(2, _NB),
    in_specs=[_pblk, _pblk, _pblk, _pblk, _pbc1, _pbc1, _pbc1, _pbcw],
    out_specs=_pout,
    out_shape=jax.ShapeDtypeStruct((_N, _D), jnp.float32),
    scratch_shapes=_layer_scratch,
)


def kernel(x, edge_index, W0, b0, g0, be0, W1, b1, g1, be1, W2, b2, g2, be2):
    src = edge_index[0]
    dst = edge_index[1]
    pad = _EPAD - _E
    spread = (jnp.arange(pad, dtype=jnp.int32) % 128)
    src2 = jnp.concatenate([src, spread]).reshape(_NW * _KCH, _CHUNK)
    dst2 = jnp.concatenate([dst, _N + spread]).reshape(_NW * _KCH, _CHUNK)
    zpad = jnp.zeros((_RPT, _D), jnp.float32)

    sc_scatter, sc_count = _sc_kernels()
    ones_nd = jnp.ones((_N, _D), jnp.float32)
    tdeg = sc_count(ones_nd, dst2, zpad)
    hs, dinv = _tc_prep(tdeg[0], tdeg[1], x, W0)

    params = ((b0, g0, be0, W1), (b1, g1, be1, W2), (b2, g2, be2, W2))
    for li, (b, g, be, Wn) in enumerate(params):
        t = sc_scatter(hs, src2, dst2, zpad)
        fused = _tc_layer_mm if li < 2 else _tc_layer_final
        res = fused(t[0], t[1], hs, dinv, b.reshape(1, _D),
                    g.reshape(1, _D), be.reshape(1, _D), Wn)
        if li < 2:
            hs = res
        else:
            out = res
    return out
